# baseline (speedup 1.0000x reference)
import functools

import jax
import jax.numpy as jnp
from jax import lax
from jax.experimental import pallas as pl
from jax.experimental.pallas import tpu as pltpu
from jax.experimental.pallas import tpu_sc as plsc

_EPS = 1e-06
_TINY = 1e-30
_NIMG = 4
_N = 4096
_D = 32
_SEG = 576
_GSEG = _NIMG * _SEG
_AUGW = 80
_NW = 32
_CHUNK = (_NIMG * _N) // _NW
_RPT = (2 * _SEG) // 16


def _rinv_kernel(pred_ref, rinv_ref):
    p = pred_ref[...]
    na2 = jnp.sum(p * p, axis=1, keepdims=True)
    rinv_ref[...] = lax.rsqrt(jnp.maximum(na2, _TINY))


def _make_seg_call():
    mesh = plsc.VectorSubcoreMesh(core_axis_name="c", subcore_axis_name="s")

    @functools.partial(
        pl.kernel,
        mesh=mesh,
        out_type=jax.ShapeDtypeStruct((2, 2 * _SEG, _AUGW), jnp.float32),
        compiler_params=pltpu.CompilerParams(use_tc_tiling_on_sc=False),
        scratch_types=[
            pltpu.VMEM((_CHUNK, _AUGW), jnp.float32),
            pltpu.VMEM((_CHUNK, _D), jnp.float32),
            pltpu.VMEM((_CHUNK,), jnp.float32),
            pltpu.VMEM((_CHUNK,), jnp.int32),
            pltpu.VMEM((_CHUNK,), jnp.int32),
            pltpu.VMEM((4, 128), jnp.int32),
            pltpu.VMEM_SHARED((2 * _SEG, _AUGW), jnp.float32),
        ],
    )
    def seg_kernel(pred_hbm, rinv_hbm, gt_hbm, an_hbm, out_hbm,
                   aug_v, pred_v, rinv_v, gt_v, an_v, idx_v,
                   table):
        c = lax.axis_index("c")
        s = lax.axis_index("s")
        w = c * 16 + s
        base = w * _CHUNK
        gbase = (s // 8) * _SEG

        iota16 = lax.iota(jnp.int32, 16)
        zvec = jnp.zeros((16,), jnp.float32)
        e0vec = jnp.where(iota16 == 0, 1.0, 0.0).astype(jnp.float32)

        def zfill(r, carry):
            for kc in range(_AUGW // 16):
                aug_v[r, pl.ds(kc * 16, 16)] = zvec
            return carry

        lax.fori_loop(0, _RPT, zfill, 0)
        pltpu.sync_copy(aug_v.at[pl.ds(0, _RPT)],
                        table.at[pl.ds(s * _RPT, _RPT)])

        pltpu.sync_copy(pred_hbm.at[pl.ds(base, _CHUNK)], pred_v)
        pltpu.sync_copy(rinv_hbm.at[pl.ds(base, _CHUNK)], rinv_v)
        pltpu.sync_copy(gt_hbm.at[pl.ds(base, _CHUNK)], gt_v)
        pltpu.sync_copy(an_hbm.at[pl.ds(base, _CHUNK)], an_v)

        for k in range(_CHUNK // 16):
            g = gt_v[pl.ds(k * 16, 16)]
            a = an_v[pl.ds(k * 16, 16)]
            idx_v[k // 8, pl.ds((k % 8) * 16, 16)] = gbase + a * 64 + g

        def prow16(r0, carry):
            rb = rinv_v[pl.ds(r0 * 16, 16)]
            for j in range(16):
                r = r0 * 16 + j
                rv = rb[j]
                v0 = pred_v[r, pl.ds(0, 16)]
                v1 = pred_v[r, pl.ds(16, 16)]
                aug_v[r, pl.ds(0, 16)] = v0
                aug_v[r, pl.ds(16, 16)] = v1
                aug_v[r, pl.ds(_D, 16)] = v0 * rv
                aug_v[r, pl.ds(_D + 16, 16)] = v1 * rv
                aug_v[r, pl.ds(2 * _D, 16)] = e0vec
            return carry

        lax.fori_loop(0, _CHUNK // 16, prow16, 0)

        plsc.subcore_barrier()

        for k in range(4):
            pltpu.sync_copy(aug_v.at[pl.ds(k * 128, 128)],
                            table.at[idx_v.at[k]], add=True)

        plsc.subcore_barrier()

        pltpu.sync_copy(table.at[pl.ds(s * _RPT, _RPT)],
                        out_hbm.at[c, pl.ds(s * _RPT, _RPT)])

    return seg_kernel


_seg_call = _make_seg_call()


def _finish_kernel(t_ref, out_ref):
    T = t_ref[...]
    S = T[:, 0:_D]
    P = T[:, _D:2 * _D]
    cnt = T[:, 2 * _D:2 * _D + 1]
    present = cnt > 0.0
    pf = present.astype(jnp.float32)
    safe = jnp.where(present, cnt, 1.0)
    S2 = jnp.sum(S * S, axis=1, keepdims=True)
    that = S * lax.rsqrt(jnp.maximum(S2, _TINY))
    pull_g = 1.0 - jnp.sum(that * P, axis=1, keepdims=True) / safe

    sel = (lax.broadcasted_iota(jnp.int32, (_NIMG * 9, _GSEG), 1) // 64
           == lax.broadcasted_iota(jnp.int32, (_NIMG * 9, _GSEG), 0)
           ).astype(jnp.float32)
    dn = (((1,), (0,)), ((), ()))
    obj = lax.dot_general(sel, pf, dn,
                          preferred_element_type=jnp.float32)
    pullnum = lax.dot_general(sel, pf * pull_g, dn,
                              preferred_element_type=jnp.float32)
    Sa = lax.dot_general(sel, pf * that, dn,
                         preferred_element_type=jnp.float32)
    els = lax.dot_general(sel, cnt, dn,
                          preferred_element_type=jnp.float32)

    Ssq = jnp.sum(Sa * Sa, axis=1, keepdims=True)
    push = (obj * obj + Ssq - 2.0 * obj) / (((obj - 1.0) * obj + _EPS) * 2.0)
    pull = pullnum / (obj + _EPS)
    la = jnp.where(obj <= 1.0, 0.0, pull + push)
    la = jnp.where(els > 0.0, la, 0.0)

    imgsel = (lax.broadcasted_iota(jnp.int32, (_NIMG, _NIMG * 9), 1) // 9
              == lax.broadcasted_iota(jnp.int32, (_NIMG, _NIMG * 9), 0)
              ).astype(jnp.float32)
    an_count = lax.dot_general(imgsel, (els > 0.0).astype(jnp.float32), dn,
                               preferred_element_type=jnp.float32)
    img_loss = lax.dot_general(imgsel, la, dn,
                               preferred_element_type=jnp.float32) / an_count
    out_ref[...] = jnp.full((1, 1), jnp.sum(img_loss) / _NIMG, jnp.float32)


def kernel(pred, gt_inds, anchor_inds):
    pred_flat = pred.reshape(_NIMG * _N, _D)
    gt_flat = gt_inds.astype(jnp.int32).reshape(-1)
    an_flat = anchor_inds.astype(jnp.int32).reshape(-1)

    rinv = pl.pallas_call(
        _rinv_kernel,
        out_shape=jax.ShapeDtypeStruct((_NIMG * _N, 1), jnp.float32),
    )(pred_flat)

    parts = _seg_call(pred_flat, rinv.reshape(-1), gt_flat, an_flat)

    out = pl.pallas_call(
        _finish_kernel,
        out_shape=jax.ShapeDtypeStruct((1, 1), jnp.float32),
    )(parts.reshape(_GSEG, _AUGW))
    return out[0, 0]


# speedup vs baseline: 1.0398x; 1.0398x over previous
import functools

import jax
import jax.numpy as jnp
from jax import lax
from jax.experimental import pallas as pl
from jax.experimental.pallas import tpu as pltpu
from jax.experimental.pallas import tpu_sc as plsc

_EPS = 1e-06
_TINY = 1e-30
_NIMG = 4
_N = 4096
_D = 32
_SEG = 576
_GSEG = _NIMG * _SEG
_AUGW = 80
_NW = 32
_CHUNK = (_NIMG * _N) // _NW
_RPT = (2 * _SEG) // 16


def _prep_kernel(pred_ref, aug_ref):
    p = pred_ref[...]
    na2 = jnp.sum(p * p, axis=1, keepdims=True)
    phat = p * lax.rsqrt(jnp.maximum(na2, _TINY))
    onecol = (lax.broadcasted_iota(jnp.int32, (_NIMG * _N, 16), 1)
              == 0).astype(jnp.float32)
    aug_ref[...] = jnp.concatenate([p, phat, onecol], axis=1)


def _make_seg_call():
    mesh = plsc.VectorSubcoreMesh(core_axis_name="c", subcore_axis_name="s")

    @functools.partial(
        pl.kernel,
        mesh=mesh,
        out_type=jax.ShapeDtypeStruct((2, 2 * _SEG, _AUGW), jnp.float32),
        compiler_params=pltpu.CompilerParams(use_tc_tiling_on_sc=False),
        scratch_types=[
            pltpu.VMEM((_CHUNK, _AUGW), jnp.float32),
            pltpu.VMEM((_CHUNK,), jnp.int32),
            pltpu.VMEM((_CHUNK,), jnp.int32),
            pltpu.VMEM((4, 128), jnp.int32),
            pltpu.VMEM((_RPT, _AUGW), jnp.float32),
            pltpu.VMEM_SHARED((2 * _SEG, _AUGW), jnp.float32),
            pltpu.SemaphoreType.DMA,
        ],
    )
    def seg_kernel(aug_hbm, gt_hbm, an_hbm, out_hbm,
                   aug_v, gt_v, an_v, idx_v, za_v, table, sem):
        c = lax.axis_index("c")
        s = lax.axis_index("s")
        w = c * 16 + s
        base = w * _CHUNK
        gbase = (s // 8) * _SEG

        iota16 = lax.iota(jnp.int32, 16)
        zvec = jnp.zeros((16,), jnp.float32)

        cp_aug = pltpu.async_copy(aug_hbm.at[pl.ds(base, _CHUNK)], aug_v, sem)
        cp_gt = pltpu.async_copy(gt_hbm.at[pl.ds(base, _CHUNK)], gt_v, sem)
        cp_an = pltpu.async_copy(an_hbm.at[pl.ds(base, _CHUNK)], an_v, sem)

        def zfill(r, carry):
            for kc in range(_AUGW // 16):
                za_v[r, pl.ds(kc * 16, 16)] = zvec
            return carry

        lax.fori_loop(0, _RPT, zfill, 0)
        pltpu.sync_copy(za_v, table.at[pl.ds(s * _RPT, _RPT)])

        cp_aug.wait()
        cp_gt.wait()
        cp_an.wait()

        for k in range(_CHUNK // 16):
            g = gt_v[pl.ds(k * 16, 16)]
            a = an_v[pl.ds(k * 16, 16)]
            idx_v[k // 8, pl.ds((k % 8) * 16, 16)] = gbase + a * 64 + g

        plsc.subcore_barrier()

        for k in range(4):
            pltpu.sync_copy(aug_v.at[pl.ds(k * 128, 128)],
                            table.at[idx_v.at[k]], add=True)

        plsc.subcore_barrier()

        pltpu.sync_copy(table.at[pl.ds(s * _RPT, _RPT)],
                        out_hbm.at[c, pl.ds(s * _RPT, _RPT)])

    return seg_kernel


_seg_call = _make_seg_call()


def _finish_kernel(t_ref, out_ref):
    T = t_ref[...]
    S = T[:, 0:_D]
    P = T[:, _D:2 * _D]
    cnt = T[:, 2 * _D:2 * _D + 1]
    present = cnt > 0.0
    pf = present.astype(jnp.float32)
    safe = jnp.where(present, cnt, 1.0)
    S2 = jnp.sum(S * S, axis=1, keepdims=True)
    that = S * lax.rsqrt(jnp.maximum(S2, _TINY))
    pull_g = 1.0 - jnp.sum(that * P, axis=1, keepdims=True) / safe

    sel = (lax.broadcasted_iota(jnp.int32, (_NIMG * 9, _GSEG), 1) // 64
           == lax.broadcasted_iota(jnp.int32, (_NIMG * 9, _GSEG), 0)
           ).astype(jnp.float32)
    dn = (((1,), (0,)), ((), ()))
    obj = lax.dot_general(sel, pf, dn,
                          preferred_element_type=jnp.float32)
    pullnum = lax.dot_general(sel, pf * pull_g, dn,
                              preferred_element_type=jnp.float32)
    Sa = lax.dot_general(sel, pf * that, dn,
                         preferred_element_type=jnp.float32)
    els = lax.dot_general(sel, cnt, dn,
                          preferred_element_type=jnp.float32)

    Ssq = jnp.sum(Sa * Sa, axis=1, keepdims=True)
    push = (obj * obj + Ssq - 2.0 * obj) / (((obj - 1.0) * obj + _EPS) * 2.0)
    pull = pullnum / (obj + _EPS)
    la = jnp.where(obj <= 1.0, 0.0, pull + push)
    la = jnp.where(els > 0.0, la, 0.0)

    imgsel = (lax.broadcasted_iota(jnp.int32, (_NIMG, _NIMG * 9), 1) // 9
              == lax.broadcasted_iota(jnp.int32, (_NIMG, _NIMG * 9), 0)
              ).astype(jnp.float32)
    an_count = lax.dot_general(imgsel, (els > 0.0).astype(jnp.float32), dn,
                               preferred_element_type=jnp.float32)
    img_loss = lax.dot_general(imgsel, la, dn,
                               preferred_element_type=jnp.float32) / an_count
    out_ref[...] = jnp.full((1, 1), jnp.sum(img_loss) / _NIMG, jnp.float32)


def kernel(pred, gt_inds, anchor_inds):
    pred_flat = pred.reshape(_NIMG * _N, _D)
    gt_flat = gt_inds.astype(jnp.int32).reshape(-1)
    an_flat = anchor_inds.astype(jnp.int32).reshape(-1)

    aug = pl.pallas_call(
        _prep_kernel,
        out_shape=jax.ShapeDtypeStruct((_NIMG * _N, _AUGW), jnp.float32),
    )(pred_flat)

    parts = _seg_call(aug, gt_flat, an_flat)

    out = pl.pallas_call(
        _finish_kernel,
        out_shape=jax.ShapeDtypeStruct((1, 1), jnp.float32),
    )(parts.reshape(_GSEG, _AUGW))
    return out[0, 0]


# speedup vs baseline: 1.3146x; 1.2643x over previous
import functools

import jax
import jax.numpy as jnp
from jax import lax
from jax.experimental import pallas as pl
from jax.experimental.pallas import tpu as pltpu
from jax.experimental.pallas import tpu_sc as plsc

_EPS = 1e-06
_TINY = 1e-30
_NIMG = 4
_N = 4096
_D = 32
_SEG = 576
_GSEG = _NIMG * _SEG
_AUGW = 80
_NW = 32
_CHUNK = (_NIMG * _N) // _NW
_RPT = (2 * _SEG) // 16


def _prep_kernel(pred_ref, aug_ref):
    p = pred_ref[...]
    na2 = jnp.sum(p * p, axis=1, keepdims=True)
    phat = p * lax.rsqrt(jnp.maximum(na2, _TINY))
    onecol = (lax.broadcasted_iota(jnp.int32, (_NIMG * _N, 16), 1)
              == 0).astype(jnp.float32)
    aug_ref[...] = jnp.concatenate([p, phat, onecol], axis=1)


def _make_seg_call():
    mesh = plsc.VectorSubcoreMesh(core_axis_name="c", subcore_axis_name="s")

    @functools.partial(
        pl.kernel,
        mesh=mesh,
        out_type=jax.ShapeDtypeStruct((2, 2 * _SEG, _AUGW), jnp.float32),
        scratch_types=[
            pltpu.VMEM((_CHUNK, _AUGW), jnp.float32),
            pltpu.VMEM((_CHUNK,), jnp.int32),
            pltpu.VMEM((_CHUNK,), jnp.int32),
            pltpu.VMEM((4, 128), jnp.int32),
            pltpu.VMEM((_RPT, _AUGW), jnp.float32),
            pltpu.VMEM_SHARED((2 * _SEG, _AUGW), jnp.float32),
            pltpu.SemaphoreType.DMA,
        ],
    )
    def seg_kernel(aug_hbm, gt_hbm, an_hbm, out_hbm,
                   aug_v, gt_v, an_v, idx_v, za_v, table, sem):
        c = lax.axis_index("c")
        s = lax.axis_index("s")
        w = c * 16 + s
        base = w * _CHUNK
        gbase = (s // 8) * _SEG

        iota16 = lax.iota(jnp.int32, 16)
        zvec = jnp.zeros((16,), jnp.float32)

        cp_aug = pltpu.async_copy(aug_hbm.at[pl.ds(base, _CHUNK)], aug_v, sem)
        cp_gt = pltpu.async_copy(gt_hbm.at[pl.ds(base, _CHUNK)], gt_v, sem)
        cp_an = pltpu.async_copy(an_hbm.at[pl.ds(base, _CHUNK)], an_v, sem)

        def zfill(r, carry):
            for kc in range(_AUGW // 16):
                za_v[r, pl.ds(kc * 16, 16)] = zvec
            return carry

        lax.fori_loop(0, _RPT, zfill, 0)
        pltpu.sync_copy(za_v, table.at[pl.ds(s * _RPT, _RPT)])

        cp_aug.wait()
        cp_gt.wait()
        cp_an.wait()

        for k in range(_CHUNK // 16):
            g = gt_v[pl.ds(k * 16, 16)]
            a = an_v[pl.ds(k * 16, 16)]
            idx_v[k // 8, pl.ds((k % 8) * 16, 16)] = gbase + a * 64 + g

        plsc.subcore_barrier()

        for k in range(4):
            pltpu.sync_copy(aug_v.at[pl.ds(k * 128, 128)],
                            table.at[idx_v.at[k]], add=True)

        plsc.subcore_barrier()

        pltpu.sync_copy(table.at[pl.ds(s * _RPT, _RPT)],
                        out_hbm.at[c, pl.ds(s * _RPT, _RPT)])

    return seg_kernel


_seg_call = _make_seg_call()


def _finish_kernel(t_ref, out_ref):
    T = t_ref[...]
    S = T[:, 0:_D]
    P = T[:, _D:2 * _D]
    cnt = T[:, 2 * _D:2 * _D + 1]
    present = cnt > 0.0
    pf = present.astype(jnp.float32)
    safe = jnp.where(present, cnt, 1.0)
    S2 = jnp.sum(S * S, axis=1, keepdims=True)
    that = S * lax.rsqrt(jnp.maximum(S2, _TINY))
    pull_g = 1.0 - jnp.sum(that * P, axis=1, keepdims=True) / safe

    sel = (lax.broadcasted_iota(jnp.int32, (_NIMG * 9, _GSEG), 1) // 64
           == lax.broadcasted_iota(jnp.int32, (_NIMG * 9, _GSEG), 0)
           ).astype(jnp.float32)
    dn = (((1,), (0,)), ((), ()))
    obj = lax.dot_general(sel, pf, dn,
                          preferred_element_type=jnp.float32)
    pullnum = lax.dot_general(sel, pf * pull_g, dn,
                              preferred_element_type=jnp.float32)
    Sa = lax.dot_general(sel, pf * that, dn,
                         preferred_element_type=jnp.float32)
    els = lax.dot_general(sel, cnt, dn,
                          preferred_element_type=jnp.float32)

    Ssq = jnp.sum(Sa * Sa, axis=1, keepdims=True)
    push = (obj * obj + Ssq - 2.0 * obj) / (((obj - 1.0) * obj + _EPS) * 2.0)
    pull = pullnum / (obj + _EPS)
    la = jnp.where(obj <= 1.0, 0.0, pull + push)
    la = jnp.where(els > 0.0, la, 0.0)

    imgsel = (lax.broadcasted_iota(jnp.int32, (_NIMG, _NIMG * 9), 1) // 9
              == lax.broadcasted_iota(jnp.int32, (_NIMG, _NIMG * 9), 0)
              ).astype(jnp.float32)
    an_count = lax.dot_general(imgsel, (els > 0.0).astype(jnp.float32), dn,
                               preferred_element_type=jnp.float32)
    img_loss = lax.dot_general(imgsel, la, dn,
                               preferred_element_type=jnp.float32) / an_count
    out_ref[...] = jnp.full((1, 1), jnp.sum(img_loss) / _NIMG, jnp.float32)


def kernel(pred, gt_inds, anchor_inds):
    pred_flat = pred.reshape(_NIMG * _N, _D)
    gt_flat = gt_inds.astype(jnp.int32).reshape(-1)
    an_flat = anchor_inds.astype(jnp.int32).reshape(-1)

    aug = pl.pallas_call(
        _prep_kernel,
        out_shape=jax.ShapeDtypeStruct((_NIMG * _N, _AUGW), jnp.float32),
    )(pred_flat)

    parts = _seg_call(aug, gt_flat, an_flat)

    out = pl.pallas_call(
        _finish_kernel,
        out_shape=jax.ShapeDtypeStruct((1, 1), jnp.float32),
    )(parts.reshape(_GSEG, _AUGW))
    return out[0, 0]


# speedup vs baseline: 1.7191x; 1.3077x over previous
import functools

import jax
import jax.numpy as jnp
from jax import lax
from jax.experimental import pallas as pl
from jax.experimental.pallas import tpu as pltpu
from jax.experimental.pallas import tpu_sc as plsc

_EPS = 1e-06
_TINY = 1e-30
_NIMG = 4
_N = 4096
_D = 32
_SEG = 576
_GSEG = _NIMG * _SEG
_NW = 32
_CHUNK = (_NIMG * _N) // _NW
_RPT = (2 * _SEG) // 16


def _pmat_kernel(pred_ref, gt_ref, an_ref, p_ref):
    pred_t = pred_ref[0]
    gt = gt_ref[0]
    an = an_ref[0]
    na2 = jnp.sum(pred_t * pred_t, axis=0, keepdims=True)
    phat_t = pred_t * lax.rsqrt(jnp.maximum(na2, _TINY))
    c = an * 64 + gt
    seg_iota = jax.lax.broadcasted_iota(jnp.int32, (_SEG, 1), 0)
    memb = (c == seg_iota).astype(jnp.float32)
    p_ref[0] = jax.lax.dot_general(
        memb, phat_t, (((1,), (1,)), ((), ())),
        preferred_element_type=jnp.float32)


def _make_seg_call():
    mesh = plsc.VectorSubcoreMesh(core_axis_name="c", subcore_axis_name="s")

    @functools.partial(
        pl.kernel,
        mesh=mesh,
        out_type=(
            jax.ShapeDtypeStruct((2, 2 * _SEG, _D), jnp.float32),
            jax.ShapeDtypeStruct((2, 2 * _SEG, 16), jnp.float32),
        ),
        scratch_types=[
            pltpu.VMEM((_CHUNK, _D), jnp.float32),
            pltpu.VMEM((128, 16), jnp.float32),
            pltpu.VMEM((_RPT, _D), jnp.float32),
            pltpu.VMEM((_CHUNK,), jnp.int32),
            pltpu.VMEM((_CHUNK,), jnp.int32),
            pltpu.VMEM((4, 128), jnp.int32),
            pltpu.VMEM_SHARED((2 * _SEG, _D), jnp.float32),
            pltpu.VMEM_SHARED((2 * _SEG, 16), jnp.float32),
            pltpu.SemaphoreType.DMA,
        ],
    )
    def seg_kernel(pred_hbm, gt_hbm, an_hbm, out_s, out_c,
                   pred_v, e0_v, za_v, gt_v, an_v, idx_v,
                   table_s, table_c, sem):
        c = lax.axis_index("c")
        s = lax.axis_index("s")
        w = c * 16 + s
        base = w * _CHUNK
        gbase = (s // 8) * _SEG

        iota16 = lax.iota(jnp.int32, 16)
        zvec = jnp.zeros((16,), jnp.float32)
        e0vec = jnp.where(iota16 == 0, 1.0, 0.0).astype(jnp.float32)

        cp_p = pltpu.async_copy(pred_hbm.at[pl.ds(base, _CHUNK)], pred_v, sem)
        cp_gt = pltpu.async_copy(gt_hbm.at[pl.ds(base, _CHUNK)], gt_v, sem)
        cp_an = pltpu.async_copy(an_hbm.at[pl.ds(base, _CHUNK)], an_v, sem)

        def zfill(r, carry):
            za_v[r, pl.ds(0, 16)] = zvec
            za_v[r, pl.ds(16, 16)] = zvec
            e0_v[r, pl.ds(0, 16)] = zvec
            return carry

        lax.fori_loop(0, _RPT, zfill, 0)
        pltpu.sync_copy(za_v, table_s.at[pl.ds(s * _RPT, _RPT)])
        pltpu.sync_copy(e0_v.at[pl.ds(0, _RPT)],
                        table_c.at[pl.ds(s * _RPT, _RPT)])

        def efill(r, carry):
            e0_v[r, pl.ds(0, 16)] = e0vec
            return carry

        lax.fori_loop(0, 128, efill, 0)

        cp_p.wait()
        cp_gt.wait()
        cp_an.wait()

        for k in range(_CHUNK // 16):
            g = gt_v[pl.ds(k * 16, 16)]
            a = an_v[pl.ds(k * 16, 16)]
            idx_v[k // 8, pl.ds((k % 8) * 16, 16)] = gbase + a * 64 + g

        plsc.subcore_barrier()

        for k in range(4):
            pltpu.sync_copy(pred_v.at[pl.ds(k * 128, 128)],
                            table_s.at[idx_v.at[k]], add=True)
            pltpu.sync_copy(e0_v, table_c.at[idx_v.at[k]], add=True)

        plsc.subcore_barrier()

        pltpu.sync_copy(table_s.at[pl.ds(s * _RPT, _RPT)],
                        out_s.at[c, pl.ds(s * _RPT, _RPT)])
        pltpu.sync_copy(table_c.at[pl.ds(s * _RPT, _RPT)],
                        out_c.at[c, pl.ds(s * _RPT, _RPT)])

    return seg_kernel


_seg_call = _make_seg_call()


def _finish_kernel(s_ref, c_ref, p_ref, out_ref):
    S = s_ref[...]
    P = p_ref[...]
    cnt = c_ref[:, 0:1]
    present = cnt > 0.0
    pf = present.astype(jnp.float32)
    safe = jnp.where(present, cnt, 1.0)
    S2 = jnp.sum(S * S, axis=1, keepdims=True)
    that = S * lax.rsqrt(jnp.maximum(S2, _TINY))
    pull_g = 1.0 - jnp.sum(that * P, axis=1, keepdims=True) / safe

    sel = (lax.broadcasted_iota(jnp.int32, (_NIMG * 9, _GSEG), 1) // 64
           == lax.broadcasted_iota(jnp.int32, (_NIMG * 9, _GSEG), 0)
           ).astype(jnp.float32)
    dn = (((1,), (0,)), ((), ()))
    obj = lax.dot_general(sel, pf, dn,
                          preferred_element_type=jnp.float32)
    pullnum = lax.dot_general(sel, pf * pull_g, dn,
                              preferred_element_type=jnp.float32)
    Sa = lax.dot_general(sel, pf * that, dn,
                         preferred_element_type=jnp.float32)
    els = lax.dot_general(sel, cnt, dn,
                          preferred_element_type=jnp.float32)

    Ssq = jnp.sum(Sa * Sa, axis=1, keepdims=True)
    push = (obj * obj + Ssq - 2.0 * obj) / (((obj - 1.0) * obj + _EPS) * 2.0)
    pull = pullnum / (obj + _EPS)
    la = jnp.where(obj <= 1.0, 0.0, pull + push)
    la = jnp.where(els > 0.0, la, 0.0)

    imgsel = (lax.broadcasted_iota(jnp.int32, (_NIMG, _NIMG * 9), 1) // 9
              == lax.broadcasted_iota(jnp.int32, (_NIMG, _NIMG * 9), 0)
              ).astype(jnp.float32)
    an_count = lax.dot_general(imgsel, (els > 0.0).astype(jnp.float32), dn,
                               preferred_element_type=jnp.float32)
    img_loss = lax.dot_general(imgsel, la, dn,
                               preferred_element_type=jnp.float32) / an_count
    out_ref[...] = jnp.full((1, 1), jnp.sum(img_loss) / _NIMG, jnp.float32)


def kernel(pred, gt_inds, anchor_inds):
    pred_flat = pred.reshape(_NIMG * _N, _D)
    gt_flat = gt_inds.astype(jnp.int32).reshape(-1)
    an_flat = anchor_inds.astype(jnp.int32).reshape(-1)

    part_s, part_c = _seg_call(pred_flat, gt_flat, an_flat)

    pred_t = jnp.transpose(pred, (0, 2, 1))
    gt3 = gt_inds.astype(jnp.int32).reshape(_NIMG, 1, _N)
    an3 = anchor_inds.astype(jnp.int32).reshape(_NIMG, 1, _N)
    pmat = pl.pallas_call(
        _pmat_kernel,
        grid=(_NIMG,),
        in_specs=[
            pl.BlockSpec((1, _D, _N), lambda i: (i, 0, 0)),
            pl.BlockSpec((1, 1, _N), lambda i: (i, 0, 0)),
            pl.BlockSpec((1, 1, _N), lambda i: (i, 0, 0)),
        ],
        out_specs=pl.BlockSpec((1, _SEG, _D), lambda i: (i, 0, 0)),
        out_shape=jax.ShapeDtypeStruct((_NIMG, _SEG, _D), jnp.float32),
    )(pred_t, gt3, an3)

    out = pl.pallas_call(
        _finish_kernel,
        out_shape=jax.ShapeDtypeStruct((1, 1), jnp.float32),
    )(part_s.reshape(_GSEG, _D), part_c.reshape(_GSEG, 16),
      pmat.reshape(_GSEG, _D))
    return out[0, 0]
